# baseline (speedup 1.0000x reference)
import functools

import jax
import jax.numpy as jnp
from jax import lax
from jax.experimental import pallas as pl
from jax.experimental.pallas import tpu as pltpu
from jax.experimental.pallas import tpu_sc as plsc

N_NODES = 10000
D = 128
NC = 2
NS = 16
NW = NC * NS
CHUNK = 64
N_PAD = 10112
ROWS_PER_SUB = N_PAD // NS


def _sc_scatter(x, src3, dst3, n_chunks):
    mesh = plsc.VectorSubcoreMesh(core_axis_name="c", subcore_axis_name="s")

    @functools.partial(
        pl.kernel,
        mesh=mesh,
        out_type=jax.ShapeDtypeStruct((NC, N_PAD, D), jnp.float32),
        scratch_types=[
            pltpu.VMEM((CHUNK, D), jnp.float32),
            pltpu.VMEM((CHUNK, D), jnp.float32),
            pltpu.VMEM((CHUNK, D), jnp.float32),
            pltpu.VMEM((CHUNK, D), jnp.float32),
            pltpu.VMEM((n_chunks // 4, CHUNK), jnp.int32),
            pltpu.VMEM((n_chunks // 4, CHUNK), jnp.int32),
            pltpu.VMEM_SHARED((N_PAD, D), jnp.float32),
            pltpu.SemaphoreType.DMA,
            pltpu.SemaphoreType.DMA,
            pltpu.SemaphoreType.DMA,
            pltpu.SemaphoreType.DMA,
        ],
    )
    def k(x_hbm, src_hbm, dst_hbm, out_hbm,
          rows0, rows1, rows2, rows3, src_v, dst_v, h_sh,
          sem0, sem1, sem2, sem3):
        c = lax.axis_index("c")
        s = lax.axis_index("s")
        wid = s * NC + c
        quarter_n = n_chunks // 4

        def zero_body(i, carry):
            r = i // (D // 16)
            col = (i % (D // 16)) * 16
            rows0[r, pl.ds(col, 16)] = jnp.zeros((16,), jnp.float32)
            return carry
        lax.fori_loop(0, CHUNK * (D // 16), zero_body, 0)
        full, rem = divmod(ROWS_PER_SUB, CHUNK)
        for t in range(full):
            pltpu.sync_copy(
                rows0, h_sh.at[pl.ds(s * ROWS_PER_SUB + t * CHUNK, CHUNK)])
        if rem:
            pltpu.sync_copy(
                rows0.at[pl.ds(0, rem)],
                h_sh.at[pl.ds(s * ROWS_PER_SUB + full * CHUNK, rem)])
        plsc.subcore_barrier()

        bufs = (rows0, rows1, rows2, rows3)
        sems = (sem0, sem1, sem2, sem3)
        for q in range(4):
            pltpu.sync_copy(src_hbm.at[wid * 4 + q], src_v)
            pltpu.sync_copy(dst_hbm.at[wid * 4 + q], dst_v)
            for t in range(3):
                pltpu.async_copy(x_hbm.at[src_v.at[t]], bufs[t], sems[t])

            def body(i, carry):
                j = i * 4
                for t in range(4):
                    buf, sem = bufs[t], sems[t]
                    nbuf, nsem = bufs[(t + 3) % 4], sems[(t + 3) % 4]
                    pltpu.make_async_copy(
                        x_hbm.at[src_v.at[0]], buf, sem).wait()
                    jnext = lax.min(j + t + 3, quarter_n - 1)
                    pltpu.async_copy(x_hbm.at[src_v.at[jnext]], nbuf, nsem)
                    pltpu.sync_copy(buf, h_sh.at[dst_v.at[j + t]], add=True)
                return carry
            lax.fori_loop(0, quarter_n // 4, body, 0)
            for t in range(3):
                pltpu.make_async_copy(
                    x_hbm.at[src_v.at[0]], bufs[t], sems[t]).wait()
        plsc.subcore_barrier()

        pltpu.sync_copy(
            h_sh.at[pl.ds(s * ROWS_PER_SUB, ROWS_PER_SUB)],
            out_hbm.at[c, pl.ds(s * ROWS_PER_SUB, ROWS_PER_SUB)])

    return k(x, src3, dst3)


def _tc_linear(parts, W, b):
    BR = 1000

    def body(p_ref, w_ref, b_ref, o_ref):
        h = p_ref[0] + p_ref[1]
        o_ref[...] = lax.dot_general(
            h, w_ref[...], (((1,), (1,)), ((), ())),
            preferred_element_type=jnp.float32) + b_ref[...]

    return pl.pallas_call(
        body,
        grid=(N_NODES // BR,),
        in_specs=[
            pl.BlockSpec((NC, BR, D), lambda i: (0, i, 0)),
            pl.BlockSpec((D, D), lambda i: (0, 0)),
            pl.BlockSpec((1, D), lambda i: (0, 0)),
        ],
        out_specs=pl.BlockSpec((BR, D), lambda i: (i, 0)),
        out_shape=jax.ShapeDtypeStruct((N_NODES, D), jnp.float32),
    )(parts, W, b.reshape(1, D))


def kernel(inputs, edge_index, W, b):
    src = edge_index[0]
    dst = edge_index[1]
    e = src.shape[0]
    n_chunks = -(-e // (NW * CHUNK))
    n_chunks += -n_chunks % 16
    e_pad = NW * CHUNK * n_chunks
    pad = e_pad - e
    pad_src = jnp.arange(pad, dtype=jnp.int32) % N_NODES
    src_p = jnp.concatenate([src.astype(jnp.int32), pad_src])
    sink = N_NODES + jnp.arange(pad, dtype=jnp.int32) % (N_PAD - N_NODES)
    dst_p = jnp.concatenate([dst.astype(jnp.int32), sink])
    src3 = src_p.reshape(NW * 4, n_chunks // 4, CHUNK)
    dst3 = dst_p.reshape(NW * 4, n_chunks // 4, CHUNK)
    parts = _sc_scatter(inputs, src3, dst3, n_chunks)
    return _tc_linear(parts, W, b)


# speedup vs baseline: 1.0204x; 1.0204x over previous
import functools

import jax
import jax.numpy as jnp
from jax import lax
from jax.experimental import pallas as pl
from jax.experimental.pallas import tpu as pltpu
from jax.experimental.pallas import tpu_sc as plsc

N_NODES = 10000
D = 128
NC = 2
NS = 16
NW = NC * NS
CHUNK = 88
N_PAD = 10112
ROWS_PER_SUB = N_PAD // NS


def _sc_scatter(x, src3, dst3, n_chunks):
    mesh = plsc.VectorSubcoreMesh(core_axis_name="c", subcore_axis_name="s")

    @functools.partial(
        pl.kernel,
        mesh=mesh,
        out_type=jax.ShapeDtypeStruct((NC, N_PAD, D), jnp.float32),
        scratch_types=[
            pltpu.VMEM((CHUNK, D), jnp.float32),
            pltpu.VMEM((CHUNK, D), jnp.float32),
            pltpu.VMEM((CHUNK, D), jnp.float32),
            pltpu.VMEM((n_chunks // 2, CHUNK), jnp.int32),
            pltpu.VMEM((n_chunks // 2, CHUNK), jnp.int32),
            pltpu.VMEM_SHARED((N_PAD, D), jnp.float32),
            pltpu.SemaphoreType.DMA,
            pltpu.SemaphoreType.DMA,
            pltpu.SemaphoreType.DMA,
        ],
    )
    def k(x_hbm, src_hbm, dst_hbm, out_hbm,
          rows0, rows1, rows2, src_v, dst_v, h_sh, sem0, sem1, sem2):
        c = lax.axis_index("c")
        s = lax.axis_index("s")
        wid = s * NC + c
        half_n = n_chunks // 2

        def zero_body(i, carry):
            r = i // (D // 16)
            col = (i % (D // 16)) * 16
            rows0[r, pl.ds(col, 16)] = jnp.zeros((16,), jnp.float32)
            return carry
        lax.fori_loop(0, CHUNK * (D // 16), zero_body, 0)
        full, rem = divmod(ROWS_PER_SUB, CHUNK)
        for t in range(full):
            pltpu.sync_copy(
                rows0, h_sh.at[pl.ds(s * ROWS_PER_SUB + t * CHUNK, CHUNK)])
        if rem:
            pltpu.sync_copy(
                rows0.at[pl.ds(0, rem)],
                h_sh.at[pl.ds(s * ROWS_PER_SUB + full * CHUNK, rem)])
        plsc.subcore_barrier()

        bufs = (rows0, rows1, rows2)
        sems = (sem0, sem1, sem2)
        for half in range(2):
            pltpu.sync_copy(src_hbm.at[wid * 2 + half], src_v)
            pltpu.sync_copy(dst_hbm.at[wid * 2 + half], dst_v)
            pltpu.async_copy(x_hbm.at[src_v.at[0]], rows0, sem0)
            pltpu.async_copy(x_hbm.at[src_v.at[1]], rows1, sem1)

            def body(i, carry):
                j = i * 3
                for t in range(3):
                    buf, sem = bufs[t], sems[t]
                    nbuf, nsem = bufs[(t + 2) % 3], sems[(t + 2) % 3]
                    pltpu.make_async_copy(
                        x_hbm.at[src_v.at[0]], buf, sem).wait()
                    jnext = lax.min(j + t + 2, half_n - 1)
                    pltpu.async_copy(x_hbm.at[src_v.at[jnext]], nbuf, nsem)
                    pltpu.sync_copy(buf, h_sh.at[dst_v.at[j + t]], add=True)
                return carry
            lax.fori_loop(0, half_n // 3, body, 0)
            pltpu.make_async_copy(x_hbm.at[src_v.at[0]], rows0, sem0).wait()
            pltpu.make_async_copy(x_hbm.at[src_v.at[0]], rows1, sem1).wait()
        plsc.subcore_barrier()

        pltpu.sync_copy(
            h_sh.at[pl.ds(s * ROWS_PER_SUB, ROWS_PER_SUB)],
            out_hbm.at[c, pl.ds(s * ROWS_PER_SUB, ROWS_PER_SUB)])

    return k(x, src3, dst3)


def _tc_linear(parts, W, b):
    BR = 1000

    def body(p_ref, w_ref, b_ref, o_ref):
        h = p_ref[0] + p_ref[1]
        o_ref[...] = lax.dot_general(
            h, w_ref[...], (((1,), (1,)), ((), ())),
            preferred_element_type=jnp.float32) + b_ref[...]

    return pl.pallas_call(
        body,
        grid=(N_NODES // BR,),
        in_specs=[
            pl.BlockSpec((NC, BR, D), lambda i: (0, i, 0)),
            pl.BlockSpec((D, D), lambda i: (0, 0)),
            pl.BlockSpec((1, D), lambda i: (0, 0)),
        ],
        out_specs=pl.BlockSpec((BR, D), lambda i: (i, 0)),
        out_shape=jax.ShapeDtypeStruct((N_NODES, D), jnp.float32),
    )(parts, W, b.reshape(1, D))


def kernel(inputs, edge_index, W, b):
    src = edge_index[0]
    dst = edge_index[1]
    e = src.shape[0]
    n_chunks = -(-e // (NW * CHUNK))
    n_chunks += -n_chunks % 6
    e_pad = NW * CHUNK * n_chunks
    pad = e_pad - e
    pad_src = jnp.arange(pad, dtype=jnp.int32) % N_NODES
    src_p = jnp.concatenate([src.astype(jnp.int32), pad_src])
    sink = N_NODES + jnp.arange(pad, dtype=jnp.int32) % (N_PAD - N_NODES)
    dst_p = jnp.concatenate([dst.astype(jnp.int32), sink])
    src3 = src_p.reshape(NW * 2, n_chunks // 2, CHUNK)
    dst3 = dst_p.reshape(NW * 2, n_chunks // 2, CHUNK)
    parts = _sc_scatter(inputs, src3, dst3, n_chunks)
    return _tc_linear(parts, W, b)
